# Initial kernel scaffold; baseline (speedup 1.0000x reference)
import jax
import jax.numpy as jnp
from jax.experimental import pallas as pl


def kernel(embedding, adj_row, adj_col, adj_val, base, ground_truth):
    raise NotImplementedError("write your pallas kernel here")


# speedup vs baseline: 3.7174x; 3.7174x over previous
import jax
import jax.numpy as jnp
from jax import lax
from jax.experimental import pallas as pl
from jax.experimental.pallas import tpu as pltpu
from jax.experimental.pallas import tpu_sc as plsc

_N = 50000
_EMB = 64
_B = 4096
_K = 32
_NNZ = 800000
_NC = 2
_NS = 16
_RPS = _N // _NC
_EBLK = _NNZ // 128
_Q = _EBLK // 2
_QK = (_Q + _NS - 1) // _NS
_ZCH = 200
_NCH = _RPS // _ZCH
_NCHK = (_NCH + _NS - 1) // _NS
_TBLK = (_B * _K) // 128

_mesh = plsc.VectorSubcoreMesh(core_axis_name="c", subcore_axis_name="s")
_sc_params = pltpu.CompilerParams(use_tc_tiling_on_sc=False,
                                  needs_layout_passes=False)


def _spmm_body(x_hbm, row_hbm, col_hbm, val_hbm, out_hbm,
               acc, zbuf, erow, ecol, eval_, lrow, grows, sem):
    c = lax.axis_index("c")
    s = lax.axis_index("s")
    lo = c * _RPS
    z16 = jnp.zeros((16,), jnp.float32)

    def _zb(r, cy):
        for cc in range(_EMB // 16):
            zbuf[r, pl.ds(cc * 16, 16)] = z16
        return cy
    lax.fori_loop(0, _ZCH, _zb, 0)

    def _zc(k, cy):
        j = s + _NS * k

        @pl.when(j < _NCH)
        def _():
            pltpu.sync_copy(zbuf, acc.at[pl.ds(j * _ZCH, _ZCH)])
        return cy
    lax.fori_loop(0, _NCHK, _zc, 0)
    plsc.subcore_barrier()

    def _batch(k, cy):
        q = s + _NS * k

        @pl.when(q < _Q)
        def _():
            pltpu.sync_copy(row_hbm.at[pl.ds(2 * q, 2)], erow)
            pltpu.sync_copy(col_hbm.at[pl.ds(2 * q, 2)], ecol)
            pltpu.sync_copy(val_hbm.at[pl.ds(2 * q, 2)], eval_)
            for i in range(2):
                pltpu.async_copy(x_hbm.at[ecol.at[i]],
                                 grows.at[pl.ds(i * 128, 128)], sem).wait()
            for i in range(2):
                def _msk(g, cy2):
                    off = pl.multiple_of(g * 16, 16)
                    rv = erow[i, pl.ds(off, 16)]
                    vv = eval_[i, pl.ds(off, 16)]
                    inr = (rv >= lo) & (rv < lo + _RPS)
                    lrow[i, pl.ds(off, 16)] = jnp.clip(rv - lo, 0, _RPS - 1)
                    eval_[i, pl.ds(off, 16)] = jnp.where(inr, vv,
                                                         jnp.float32(0.0))
                    return cy2
                lax.fori_loop(0, 8, _msk, 0)

                def _edge(g, cy2):
                    off = pl.multiple_of(g * 16, 16)
                    vv = eval_[i, pl.ds(off, 16)]
                    for j in range(16):
                        vm = vv[j]
                        r = i * 128 + off + j
                        for cc in range(_EMB // 16):
                            sl = pl.ds(cc * 16, 16)
                            grows[r, sl] = grows[r, sl] * vm
                    return cy2
                lax.fori_loop(0, 8, _edge, 0)
                pltpu.sync_copy(grows.at[pl.ds(i * 128, 128)],
                                acc.at[lrow.at[i]], add=True)
        return cy
    lax.fori_loop(0, _QK, _batch, 0)
    plsc.subcore_barrier()

    def _wo(k, cy):
        j = s + _NS * k

        @pl.when(j < _NCH)
        def _():
            pltpu.sync_copy(acc.at[pl.ds(j * _ZCH, _ZCH)],
                            out_hbm.at[pl.ds(lo + j * _ZCH, _ZCH)])
        return cy
    lax.fori_loop(0, _NCHK, _wo, 0)


def _spmm(x, row2, col2, val2):
    f = pl.kernel(
        _spmm_body,
        out_type=jax.ShapeDtypeStruct((_N, _EMB), jnp.float32),
        mesh=_mesh,
        scratch_types=[
            pltpu.VMEM_SHARED((_RPS, _EMB), jnp.float32),
            pltpu.VMEM((_ZCH, _EMB), jnp.float32),
            pltpu.VMEM((2, 128), jnp.int32),
            pltpu.VMEM((2, 128), jnp.int32),
            pltpu.VMEM((2, 128), jnp.float32),
            pltpu.VMEM((2, 128), jnp.int32),
            pltpu.VMEM((256, _EMB), jnp.float32),
            pltpu.SemaphoreType.DMA,
        ],
        compiler_params=_sc_params,
    )
    return f(x, row2, col2, val2)


def _dec_body(hyper, emb, hidx, ridx, tidx, out_hbm,
              ihb, irb, itb, gh, gr, gt, sbuf, sem):
    c = lax.axis_index("c")
    s = lax.axis_index("s")
    wid = s * _NC + c

    def _batch(b, cy):
        blk = wid * 32 + 2 * b
        pltpu.sync_copy(hidx.at[pl.ds(blk, 2)], ihb)
        pltpu.sync_copy(ridx.at[pl.ds(blk, 2)], irb)
        pltpu.sync_copy(tidx.at[pl.ds(blk, 2)], itb)
        for i in range(2):
            pltpu.async_copy(hyper.at[ihb.at[i]],
                             gh.at[pl.ds(i * 128, 128)], sem).wait()
            pltpu.async_copy(emb.at[irb.at[i]],
                             gr.at[pl.ds(i * 128, 128)], sem).wait()
            pltpu.async_copy(hyper.at[itb.at[i]],
                             gt.at[pl.ds(i * 128, 128)], sem).wait()

        lanes = lax.iota(jnp.int32, 16)

        def _grp(g, cy2):
            off = pl.multiple_of(g * 16, 16)
            svec = jnp.zeros((16,), jnp.float32)
            for j in range(16):
                e = off + j
                acc = jnp.zeros((16,), jnp.float32)
                for cc in range(_EMB // 32):
                    slre = pl.ds(cc * 16, 16)
                    slim = pl.ds(_EMB // 2 + cc * 16, 16)
                    hr = gh[e, slre]
                    hi = gh[e, slim]
                    rr = gr[e, slre]
                    ri = gr[e, slim]
                    tr = gt[e, slre]
                    ti = gt[e, slim]
                    acc = (acc + (hr * rr - hi * ri) * tr
                           + (hr * ri + hi * rr) * ti)
                tot = jnp.sum(acc)
                svec = jnp.where(lanes == j, tot, svec)
            sbuf[pl.ds(off, 16)] = svec
            return cy2
        lax.fori_loop(0, 16, _grp, 0)
        pltpu.sync_copy(sbuf, out_hbm.at[pl.ds(blk * 128, 256)])
        return cy
    lax.fori_loop(0, 16, _batch, 0)


def _dec(hyper, emb, hidx, ridx, tidx):
    f = pl.kernel(
        _dec_body,
        out_type=jax.ShapeDtypeStruct((_B * _K,), jnp.float32),
        mesh=_mesh,
        scratch_types=[
            pltpu.VMEM((2, 128), jnp.int32),
            pltpu.VMEM((2, 128), jnp.int32),
            pltpu.VMEM((2, 128), jnp.int32),
            pltpu.VMEM((256, _EMB), jnp.float32),
            pltpu.VMEM((256, _EMB), jnp.float32),
            pltpu.VMEM((256, _EMB), jnp.float32),
            pltpu.VMEM((256,), jnp.float32),
            pltpu.SemaphoreType.DMA,
        ],
        compiler_params=_sc_params,
    )
    return f(hyper, emb, hidx, ridx, tidx)


def _avg_body(a_ref, b_ref, c_ref, o_ref):
    o_ref[...] = (a_ref[...] + b_ref[...] + c_ref[...]) * jnp.float32(1.0 / 3.0)


def _avg3(a, b, c):
    g = 25
    blk = _N // g
    return pl.pallas_call(
        _avg_body,
        out_shape=jax.ShapeDtypeStruct((_N, _EMB), jnp.float32),
        grid=(g,),
        in_specs=[pl.BlockSpec((blk, _EMB), lambda i: (i, 0))] * 3,
        out_specs=pl.BlockSpec((blk, _EMB), lambda i: (i, 0)),
    )(a, b, c)


def _loss_body(sc_ref, gt_ref, emb_ref, o_ref):
    sv = jax.nn.sigmoid(sc_ref[...])
    gv = gt_ref[...]
    eps = jnp.float32(1e-7)
    bce = -jnp.mean(gv * jnp.log(sv + eps)
                    + (1.0 - gv) * jnp.log(1.0 - sv + eps))
    ev = emb_ref[...]
    ae = jnp.abs(ev)
    regu = jnp.float32(0.01) * jnp.sum(ae * ae * ae)
    o_ref[0, 0] = bce + regu


def _loss(scores, gt, emb):
    return pl.pallas_call(
        _loss_body,
        out_shape=jax.ShapeDtypeStruct((1, 1), jnp.float32),
        out_specs=pl.BlockSpec(memory_space=pltpu.SMEM),
    )(scores, gt, emb)


def kernel(embedding, adj_row, adj_col, adj_val, base, ground_truth):
    emb = embedding.astype(jnp.float32)
    row2 = adj_row.astype(jnp.int32).reshape(_EBLK, 128)
    col2 = adj_col.astype(jnp.int32).reshape(_EBLK, 128)
    val2 = adj_val.astype(jnp.float32).reshape(_EBLK, 128)
    x1 = _spmm(emb, row2, col2, val2)
    x2 = _spmm(x1, row2, col2, val2)
    hyper = _avg3(emb, x1, x2)
    b32 = base.astype(jnp.int32)
    hidx = b32[:, :_K].reshape(_TBLK, 128)
    ridx = b32[:, _K:2 * _K].reshape(_TBLK, 128)
    tidx = b32[:, 2 * _K:].reshape(_TBLK, 128)
    scores = _dec(hyper, emb, hidx, ridx, tidx)
    loss = _loss(scores.reshape(_B, _K), ground_truth.astype(jnp.float32), emb)
    return loss[0, 0]


# speedup vs baseline: 6.3026x; 1.6954x over previous
import jax
import jax.numpy as jnp
from jax import lax
from jax.experimental import pallas as pl
from jax.experimental.pallas import tpu as pltpu
from jax.experimental.pallas import tpu_sc as plsc

_N = 50000
_EMB = 64
_B = 4096
_K = 32
_NNZ = 800000
_NC = 2
_NS = 16
_RPS = _N // _NC
_EBLK = _NNZ // 128
_QK = (_EBLK + _NS - 1) // _NS
_QK2 = (_QK + 1) // 2
_ZCH = 200
_NCH = _RPS // _ZCH
_NCHK = (_NCH + _NS - 1) // _NS
_TBLK = (_B * _K) // 128

_mesh = plsc.VectorSubcoreMesh(core_axis_name="c", subcore_axis_name="s")
_sc_params = pltpu.CompilerParams(use_tc_tiling_on_sc=False,
                                  needs_layout_passes=False)


def _spmm_body(x_hbm, row_hbm, col_hbm, val_hbm, out_hbm,
               acc, erow, ecol, eval_, lrow, grows,
               msem0, msem1, gsem0, gsem1, ssem0, ssem1, zsem):
    c = lax.axis_index("c")
    s = lax.axis_index("s")
    lo = c * _RPS
    msem = (msem0, msem1)
    gsem = (gsem0, gsem1)
    ssem = (ssem0, ssem1)
    z16 = jnp.zeros((16,), jnp.float32)

    def _zb(r, cy):
        for cc in range(_EMB // 16):
            grows[r, pl.ds(cc * 16, 16)] = z16
        return cy
    lax.fori_loop(0, _ZCH, _zb, 0)

    def _zc(k, cy):
        j = s + _NS * k

        @pl.when(j < _NCH)
        def _():
            pltpu.async_copy(grows.at[pl.ds(0, _ZCH)],
                             acc.at[pl.ds(j * _ZCH, _ZCH)], zsem)
        return cy
    lax.fori_loop(0, _NCHK, _zc, 0)

    def _zw(k, cy):
        j = s + _NS * k

        @pl.when(j < _NCH)
        def _():
            pltpu.make_async_copy(grows.at[pl.ds(0, _ZCH)],
                                  acc.at[pl.ds(j * _ZCH, _ZCH)], zsem).wait()
        return cy
    lax.fori_loop(0, _NCHK, _zw, 0)
    plsc.subcore_barrier()

    def _q(k):
        return s + _NS * k

    def _start_meta(k, b):
        @pl.when(_q(k) < _EBLK)
        def _():
            q = _q(k)
            pltpu.async_copy(row_hbm.at[pl.ds(q, 1)],
                             erow.at[pl.ds(b, 1)], msem[b])
            pltpu.async_copy(col_hbm.at[pl.ds(q, 1)],
                             ecol.at[pl.ds(b, 1)], msem[b])
            pltpu.async_copy(val_hbm.at[pl.ds(q, 1)],
                             eval_.at[pl.ds(b, 1)], msem[b])

    def _wait_meta(k, b):
        @pl.when(_q(k) < _EBLK)
        def _():
            pltpu.make_async_copy(row_hbm.at[pl.ds(0, 1)],
                                  erow.at[pl.ds(b, 1)], msem[b]).wait()
            pltpu.make_async_copy(col_hbm.at[pl.ds(0, 1)],
                                  ecol.at[pl.ds(b, 1)], msem[b]).wait()
            pltpu.make_async_copy(val_hbm.at[pl.ds(0, 1)],
                                  eval_.at[pl.ds(b, 1)], msem[b]).wait()

    def _start_gather(k, b):
        @pl.when(_q(k) < _EBLK)
        def _():
            pltpu.async_copy(x_hbm.at[ecol.at[b]],
                             grows.at[pl.ds(b * 128, 128)], gsem[b])

    def _wait_gather(k, b):
        @pl.when(_q(k) < _EBLK)
        def _():
            pltpu.make_async_copy(
                x_hbm.at[pl.ds(0, 128)],
                grows.at[pl.ds(b * 128, 128)], gsem[b]).wait()

    def _compute(k, b):
        @pl.when(_q(k) < _EBLK)
        def _():
            def _msk(g, cy2):
                off = pl.multiple_of(g * 16, 16)
                rv = erow[b, pl.ds(off, 16)]
                vv = eval_[b, pl.ds(off, 16)]
                inr = (rv >= lo) & (rv < lo + _RPS)
                lrow[b, pl.ds(off, 16)] = jnp.clip(rv - lo, 0, _RPS - 1)
                eval_[b, pl.ds(off, 16)] = jnp.where(inr, vv,
                                                     jnp.float32(0.0))
                return cy2
            lax.fori_loop(0, 8, _msk, 0)

            def _edge(g, cy2):
                off = pl.multiple_of(g * 16, 16)
                vv = eval_[b, pl.ds(off, 16)]
                for j in range(16):
                    vm = vv[j]
                    r = b * 128 + off + j
                    for cc in range(_EMB // 16):
                        sl = pl.ds(cc * 16, 16)
                        grows[r, sl] = grows[r, sl] * vm
                return cy2
            lax.fori_loop(0, 8, _edge, 0)

    def _start_scatter(k, b):
        @pl.when(_q(k) < _EBLK)
        def _():
            pltpu.async_copy(grows.at[pl.ds(b * 128, 128)],
                             acc.at[lrow.at[b]], ssem[b], add=True)

    def _wait_scatter(k, b):
        @pl.when(jnp.logical_and(k >= 0, _q(k) < _EBLK))
        def _():
            pltpu.make_async_copy(
                x_hbm.at[pl.ds(0, 128)],
                grows.at[pl.ds(b * 128, 128)], ssem[b]).wait()

    _start_meta(0, 0)
    _start_meta(1, 1)
    _wait_meta(0, 0)
    _start_gather(0, 0)

    def _body(t, cy):
        for b in range(2):
            k = 2 * t + b
            nb = 1 - b
            _wait_gather(k, b)
            _wait_meta(k + 1, nb)
            _wait_scatter(k - 1, nb)
            _start_gather(k + 1, nb)
            _compute(k, b)
            _start_meta(k + 2, b)
            _start_scatter(k, b)
        return cy
    lax.fori_loop(0, _QK2, _body, 0)
    _wait_scatter(2 * _QK2 - 1, 1)
    plsc.subcore_barrier()

    def _wo(k, cy):
        j = s + _NS * k

        @pl.when(j < _NCH)
        def _():
            pltpu.async_copy(acc.at[pl.ds(j * _ZCH, _ZCH)],
                             out_hbm.at[pl.ds(lo + j * _ZCH, _ZCH)], zsem)
        return cy
    lax.fori_loop(0, _NCHK, _wo, 0)

    def _wow(k, cy):
        j = s + _NS * k

        @pl.when(j < _NCH)
        def _():
            pltpu.make_async_copy(acc.at[pl.ds(j * _ZCH, _ZCH)],
                                  out_hbm.at[pl.ds(lo + j * _ZCH, _ZCH)],
                                  zsem).wait()
        return cy
    lax.fori_loop(0, _NCHK, _wow, 0)


def _spmm(x, row2, col2, val2):
    f = pl.kernel(
        _spmm_body,
        out_type=jax.ShapeDtypeStruct((_N, _EMB), jnp.float32),
        mesh=_mesh,
        scratch_types=[
            pltpu.VMEM_SHARED((_RPS, _EMB), jnp.float32),
            pltpu.VMEM((2, 128), jnp.int32),
            pltpu.VMEM((2, 128), jnp.int32),
            pltpu.VMEM((2, 128), jnp.float32),
            pltpu.VMEM((2, 128), jnp.int32),
            pltpu.VMEM((256, _EMB), jnp.float32),
            pltpu.SemaphoreType.DMA,
            pltpu.SemaphoreType.DMA,
            pltpu.SemaphoreType.DMA,
            pltpu.SemaphoreType.DMA,
            pltpu.SemaphoreType.DMA,
            pltpu.SemaphoreType.DMA,
            pltpu.SemaphoreType.DMA,
        ],
        compiler_params=_sc_params,
    )
    return f(x, row2, col2, val2)


def _dec_body(hyper, emb, hidx, ridx, tidx, out_hbm,
              ihb, irb, itb, gh, gr, gt, sbuf, sem):
    c = lax.axis_index("c")
    s = lax.axis_index("s")
    wid = s * _NC + c

    def _batch(b, cy):
        blk = wid * 32 + 2 * b
        pltpu.sync_copy(hidx.at[pl.ds(blk, 2)], ihb)
        pltpu.sync_copy(ridx.at[pl.ds(blk, 2)], irb)
        pltpu.sync_copy(tidx.at[pl.ds(blk, 2)], itb)
        for i in range(2):
            pltpu.async_copy(hyper.at[ihb.at[i]],
                             gh.at[pl.ds(i * 128, 128)], sem).wait()
            pltpu.async_copy(emb.at[irb.at[i]],
                             gr.at[pl.ds(i * 128, 128)], sem).wait()
            pltpu.async_copy(hyper.at[itb.at[i]],
                             gt.at[pl.ds(i * 128, 128)], sem).wait()

        lanes = lax.iota(jnp.int32, 16)

        def _grp(g, cy2):
            off = pl.multiple_of(g * 16, 16)
            svec = jnp.zeros((16,), jnp.float32)
            for j in range(16):
                e = off + j
                acc = jnp.zeros((16,), jnp.float32)
                for cc in range(_EMB // 32):
                    slre = pl.ds(cc * 16, 16)
                    slim = pl.ds(_EMB // 2 + cc * 16, 16)
                    hr = gh[e, slre]
                    hi = gh[e, slim]
                    rr = gr[e, slre]
                    ri = gr[e, slim]
                    tr = gt[e, slre]
                    ti = gt[e, slim]
                    acc = (acc + (hr * rr - hi * ri) * tr
                           + (hr * ri + hi * rr) * ti)
                tot = jnp.sum(acc)
                svec = jnp.where(lanes == j, tot, svec)
            sbuf[pl.ds(off, 16)] = svec
            return cy2
        lax.fori_loop(0, 16, _grp, 0)
        pltpu.sync_copy(sbuf, out_hbm.at[pl.ds(blk * 128, 256)])
        return cy
    lax.fori_loop(0, 16, _batch, 0)


def _dec(hyper, emb, hidx, ridx, tidx):
    f = pl.kernel(
        _dec_body,
        out_type=jax.ShapeDtypeStruct((_B * _K,), jnp.float32),
        mesh=_mesh,
        scratch_types=[
            pltpu.VMEM((2, 128), jnp.int32),
            pltpu.VMEM((2, 128), jnp.int32),
            pltpu.VMEM((2, 128), jnp.int32),
            pltpu.VMEM((256, _EMB), jnp.float32),
            pltpu.VMEM((256, _EMB), jnp.float32),
            pltpu.VMEM((256, _EMB), jnp.float32),
            pltpu.VMEM((256,), jnp.float32),
            pltpu.SemaphoreType.DMA,
        ],
        compiler_params=_sc_params,
    )
    return f(hyper, emb, hidx, ridx, tidx)


def _avg_body(a_ref, b_ref, c_ref, o_ref):
    o_ref[...] = (a_ref[...] + b_ref[...] + c_ref[...]) * jnp.float32(1.0 / 3.0)


def _avg3(a, b, c):
    g = 25
    blk = _N // g
    return pl.pallas_call(
        _avg_body,
        out_shape=jax.ShapeDtypeStruct((_N, _EMB), jnp.float32),
        grid=(g,),
        in_specs=[pl.BlockSpec((blk, _EMB), lambda i: (i, 0))] * 3,
        out_specs=pl.BlockSpec((blk, _EMB), lambda i: (i, 0)),
    )(a, b, c)


def _loss_body(sc_ref, gt_ref, emb_ref, o_ref):
    sv = jax.nn.sigmoid(sc_ref[...])
    gv = gt_ref[...]
    eps = jnp.float32(1e-7)
    bce = -jnp.mean(gv * jnp.log(sv + eps)
                    + (1.0 - gv) * jnp.log(1.0 - sv + eps))
    ev = emb_ref[...]
    ae = jnp.abs(ev)
    regu = jnp.float32(0.01) * jnp.sum(ae * ae * ae)
    o_ref[0, 0] = bce + regu


def _loss(scores, gt, emb):
    return pl.pallas_call(
        _loss_body,
        out_shape=jax.ShapeDtypeStruct((1, 1), jnp.float32),
        out_specs=pl.BlockSpec(memory_space=pltpu.SMEM),
    )(scores, gt, emb)


def kernel(embedding, adj_row, adj_col, adj_val, base, ground_truth):
    emb = embedding.astype(jnp.float32)
    row2 = adj_row.astype(jnp.int32).reshape(_EBLK, 128)
    col2 = adj_col.astype(jnp.int32).reshape(_EBLK, 128)
    val2 = adj_val.astype(jnp.float32).reshape(_EBLK, 128)
    x1 = _spmm(emb, row2, col2, val2)
    x2 = _spmm(x1, row2, col2, val2)
    hyper = _avg3(emb, x1, x2)
    b32 = base.astype(jnp.int32)
    hidx = b32[:, :_K].reshape(_TBLK, 128)
    ridx = b32[:, _K:2 * _K].reshape(_TBLK, 128)
    tidx = b32[:, 2 * _K:].reshape(_TBLK, 128)
    scores = _dec(hyper, emb, hidx, ridx, tidx)
    loss = _loss(scores.reshape(_B, _K), ground_truth.astype(jnp.float32), emb)
    return loss[0, 0]


# speedup vs baseline: 8.4337x; 1.3381x over previous
import jax
import jax.numpy as jnp
from jax import lax
from jax.experimental import pallas as pl
from jax.experimental.pallas import tpu as pltpu
from jax.experimental.pallas import tpu_sc as plsc

_N = 50000
_EMB = 64
_B = 4096
_K = 32
_NNZ = 800000
_NC = 2
_NS = 16
_RPS = _N // _NC
_EBLK = _NNZ // 128
_QK = (_EBLK + _NS - 1) // _NS
_QK2 = (_QK + 1) // 2
_ZCH = 200
_NCH = _RPS // _ZCH
_NCHK = (_NCH + _NS - 1) // _NS
_TBLK = (_B * _K) // 128

_mesh = plsc.VectorSubcoreMesh(core_axis_name="c", subcore_axis_name="s")
_sc_params = pltpu.CompilerParams(use_tc_tiling_on_sc=False,
                                  needs_layout_passes=False)


def _spmm_body(x_hbm, row_hbm, col_hbm, val_hbm, out_hbm,
               acc, erow, ecol, eval_, lrow, grows,
               msem0, msem1, gsem0, gsem1, ssem0, ssem1, zsem):
    c = lax.axis_index("c")
    s = lax.axis_index("s")
    lo = c * _RPS
    msem = (msem0, msem1)
    gsem = (gsem0, gsem1)
    ssem = (ssem0, ssem1)
    z32 = jnp.zeros((32,), jnp.bfloat16)

    def _zb(r, cy):
        for cc in range(_EMB // 32):
            grows[r, pl.ds(cc * 32, 32)] = z32
        return cy
    lax.fori_loop(0, _ZCH, _zb, 0)

    def _zc(k, cy):
        j = s + _NS * k

        @pl.when(j < _NCH)
        def _():
            pltpu.async_copy(grows.at[pl.ds(0, _ZCH)],
                             acc.at[pl.ds(j * _ZCH, _ZCH)], zsem)
        return cy
    lax.fori_loop(0, _NCHK, _zc, 0)

    def _zw(k, cy):
        j = s + _NS * k

        @pl.when(j < _NCH)
        def _():
            pltpu.make_async_copy(grows.at[pl.ds(0, _ZCH)],
                                  acc.at[pl.ds(j * _ZCH, _ZCH)], zsem).wait()
        return cy
    lax.fori_loop(0, _NCHK, _zw, 0)
    plsc.subcore_barrier()

    def _q(k):
        return s + _NS * k

    def _start_meta(k, b):
        @pl.when(_q(k) < _EBLK)
        def _():
            q = _q(k)
            pltpu.async_copy(row_hbm.at[pl.ds(q, 1)],
                             erow.at[pl.ds(b, 1)], msem[b])
            pltpu.async_copy(col_hbm.at[pl.ds(q, 1)],
                             ecol.at[pl.ds(b, 1)], msem[b])
            pltpu.async_copy(val_hbm.at[pl.ds(q, 1)],
                             eval_.at[pl.ds(b, 1)], msem[b])

    def _wait_meta(k, b):
        @pl.when(_q(k) < _EBLK)
        def _():
            pltpu.make_async_copy(row_hbm.at[pl.ds(0, 1)],
                                  erow.at[pl.ds(b, 1)], msem[b]).wait()
            pltpu.make_async_copy(col_hbm.at[pl.ds(0, 1)],
                                  ecol.at[pl.ds(b, 1)], msem[b]).wait()
            pltpu.make_async_copy(val_hbm.at[pl.ds(0, 1)],
                                  eval_.at[pl.ds(b, 1)], msem[b]).wait()

    def _start_gather(k, b):
        @pl.when(_q(k) < _EBLK)
        def _():
            pltpu.async_copy(x_hbm.at[ecol.at[b]],
                             grows.at[pl.ds(b * 128, 128)], gsem[b])

    def _wait_gather(k, b):
        @pl.when(_q(k) < _EBLK)
        def _():
            pltpu.make_async_copy(
                x_hbm.at[pl.ds(0, 128)],
                grows.at[pl.ds(b * 128, 128)], gsem[b]).wait()

    def _compute(k, b):
        @pl.when(_q(k) < _EBLK)
        def _():
            def _msk(g, cy2):
                off = pl.multiple_of(g * 16, 16)
                rv = erow[b, pl.ds(off, 16)]
                vv = eval_[b, pl.ds(off, 16)]
                inr = (rv >= lo) & (rv < lo + _RPS)
                lrow[b, pl.ds(off, 16)] = jnp.clip(rv - lo, 0, _RPS - 1)
                eval_[b, pl.ds(off, 16)] = jnp.where(inr, vv, 0)
                return cy2
            lax.fori_loop(0, 8, _msk, 0)

            def _edge(g, cy2):
                off = pl.multiple_of(g * 16, 16)
                vv = eval_[b, pl.ds(off, 16)]
                for j in range(16):
                    vmv = plsc.bitcast(jnp.full((16,), vv[j], jnp.int32),
                                       jnp.bfloat16)
                    r = b * 128 + off + j
                    for cc in range(_EMB // 32):
                        sl = pl.ds(cc * 32, 32)
                        grows[r, sl] = grows[r, sl] * vmv
                return cy2
            lax.fori_loop(0, 8, _edge, 0)

    def _start_scatter(k, b):
        @pl.when(_q(k) < _EBLK)
        def _():
            pltpu.async_copy(grows.at[pl.ds(b * 128, 128)],
                             acc.at[lrow.at[b]], ssem[b], add=True)

    def _wait_scatter(k, b):
        @pl.when(jnp.logical_and(k >= 0, _q(k) < _EBLK))
        def _():
            pltpu.make_async_copy(
                x_hbm.at[pl.ds(0, 128)],
                grows.at[pl.ds(b * 128, 128)], ssem[b]).wait()

    _start_meta(0, 0)
    _start_meta(1, 1)
    _wait_meta(0, 0)
    _start_gather(0, 0)

    def _body(t, cy):
        for b in range(2):
            k = 2 * t + b
            nb = 1 - b
            _wait_gather(k, b)
            _wait_meta(k + 1, nb)
            _wait_scatter(k - 1, nb)
            _start_gather(k + 1, nb)
            _compute(k, b)
            _start_meta(k + 2, b)
            _start_scatter(k, b)
        return cy
    lax.fori_loop(0, _QK2, _body, 0)
    _wait_scatter(2 * _QK2 - 1, 1)
    plsc.subcore_barrier()

    def _wo(k, cy):
        j = s + _NS * k

        @pl.when(j < _NCH)
        def _():
            pltpu.async_copy(acc.at[pl.ds(j * _ZCH, _ZCH)],
                             out_hbm.at[pl.ds(lo + j * _ZCH, _ZCH)], zsem)
        return cy
    lax.fori_loop(0, _NCHK, _wo, 0)

    def _wow(k, cy):
        j = s + _NS * k

        @pl.when(j < _NCH)
        def _():
            pltpu.make_async_copy(acc.at[pl.ds(j * _ZCH, _ZCH)],
                                  out_hbm.at[pl.ds(lo + j * _ZCH, _ZCH)],
                                  zsem).wait()
        return cy
    lax.fori_loop(0, _NCHK, _wow, 0)


def _spmm(x, row2, col2, val2):
    f = pl.kernel(
        _spmm_body,
        out_type=jax.ShapeDtypeStruct((_N, _EMB), jnp.bfloat16),
        mesh=_mesh,
        scratch_types=[
            pltpu.VMEM_SHARED((_RPS, _EMB), jnp.bfloat16),
            pltpu.VMEM((2, 128), jnp.int32),
            pltpu.VMEM((2, 128), jnp.int32),
            pltpu.VMEM((2, 128), jnp.int32),
            pltpu.VMEM((2, 128), jnp.int32),
            pltpu.VMEM((256, _EMB), jnp.bfloat16),
            pltpu.SemaphoreType.DMA,
            pltpu.SemaphoreType.DMA,
            pltpu.SemaphoreType.DMA,
            pltpu.SemaphoreType.DMA,
            pltpu.SemaphoreType.DMA,
            pltpu.SemaphoreType.DMA,
            pltpu.SemaphoreType.DMA,
        ],
        compiler_params=_sc_params,
    )
    return f(x, row2, col2, val2)


def _dec_body(hyper, emb, hidx, ridx, tidx, out_hbm,
              ihb, irb, itb, gh, gr, gt, sbuf, sem):
    c = lax.axis_index("c")
    s = lax.axis_index("s")
    wid = s * _NC + c

    def _batch(b, cy):
        blk = wid * 32 + 2 * b
        pltpu.sync_copy(hidx.at[pl.ds(blk, 2)], ihb)
        pltpu.sync_copy(ridx.at[pl.ds(blk, 2)], irb)
        pltpu.sync_copy(tidx.at[pl.ds(blk, 2)], itb)
        for i in range(2):
            pltpu.async_copy(hyper.at[ihb.at[i]],
                             gh.at[pl.ds(i * 128, 128)], sem).wait()
            pltpu.async_copy(emb.at[irb.at[i]],
                             gr.at[pl.ds(i * 128, 128)], sem).wait()
            pltpu.async_copy(hyper.at[itb.at[i]],
                             gt.at[pl.ds(i * 128, 128)], sem).wait()

        lanes = lax.iota(jnp.int32, 16)

        def _grp(g, cy2):
            off = pl.multiple_of(g * 16, 16)
            svec = jnp.zeros((16,), jnp.float32)
            for j in range(16):
                e = off + j
                acc = jnp.zeros((16,), jnp.float32)
                for cc in range(_EMB // 32):
                    slre = pl.ds(cc * 16, 16)
                    slim = pl.ds(_EMB // 2 + cc * 16, 16)
                    hr = gh[e, slre]
                    hi = gh[e, slim]
                    rr = gr[e, slre]
                    ri = gr[e, slim]
                    tr = gt[e, slre]
                    ti = gt[e, slim]
                    acc = (acc + (hr * rr - hi * ri) * tr
                           + (hr * ri + hi * rr) * ti)
                tot = jnp.sum(acc)
                svec = jnp.where(lanes == j, tot, svec)
            sbuf[pl.ds(off, 16)] = svec
            return cy2
        lax.fori_loop(0, 16, _grp, 0)
        pltpu.sync_copy(sbuf, out_hbm.at[pl.ds(blk * 128, 256)])
        return cy
    lax.fori_loop(0, 16, _batch, 0)


def _dec(hyper, emb, hidx, ridx, tidx):
    f = pl.kernel(
        _dec_body,
        out_type=jax.ShapeDtypeStruct((_B * _K,), jnp.float32),
        mesh=_mesh,
        scratch_types=[
            pltpu.VMEM((2, 128), jnp.int32),
            pltpu.VMEM((2, 128), jnp.int32),
            pltpu.VMEM((2, 128), jnp.int32),
            pltpu.VMEM((256, _EMB), jnp.float32),
            pltpu.VMEM((256, _EMB), jnp.float32),
            pltpu.VMEM((256, _EMB), jnp.float32),
            pltpu.VMEM((256,), jnp.float32),
            pltpu.SemaphoreType.DMA,
        ],
        compiler_params=_sc_params,
    )
    return f(hyper, emb, hidx, ridx, tidx)


def _avg_body(a_ref, b_ref, c_ref, o_ref):
    o_ref[...] = ((a_ref[...].astype(jnp.float32)
                   + b_ref[...].astype(jnp.float32)
                   + c_ref[...].astype(jnp.float32))
                  * jnp.float32(1.0 / 3.0))


def _avg3(a, b, c):
    g = 25
    blk = _N // g
    return pl.pallas_call(
        _avg_body,
        out_shape=jax.ShapeDtypeStruct((_N, _EMB), jnp.float32),
        grid=(g,),
        in_specs=[pl.BlockSpec((blk, _EMB), lambda i: (i, 0))] * 3,
        out_specs=pl.BlockSpec((blk, _EMB), lambda i: (i, 0)),
    )(a, b, c)


def _loss_body(sc_ref, gt_ref, emb_ref, o_ref):
    sv = jax.nn.sigmoid(sc_ref[...])
    gv = gt_ref[...]
    eps = jnp.float32(1e-7)
    bce = -jnp.mean(gv * jnp.log(sv + eps)
                    + (1.0 - gv) * jnp.log(1.0 - sv + eps))
    ev = emb_ref[...]
    ae = jnp.abs(ev)
    regu = jnp.float32(0.01) * jnp.sum(ae * ae * ae)
    o_ref[0, 0] = bce + regu


def _loss(scores, gt, emb):
    return pl.pallas_call(
        _loss_body,
        out_shape=jax.ShapeDtypeStruct((1, 1), jnp.float32),
        out_specs=pl.BlockSpec(memory_space=pltpu.SMEM),
    )(scores, gt, emb)


def kernel(embedding, adj_row, adj_col, adj_val, base, ground_truth):
    emb = embedding.astype(jnp.float32)
    row2 = adj_row.astype(jnp.int32).reshape(_EBLK, 128)
    col2 = adj_col.astype(jnp.int32).reshape(_EBLK, 128)
    vbits = jax.lax.bitcast_convert_type(adj_val.astype(jnp.bfloat16),
                                         jnp.uint16).astype(jnp.uint32)
    val2 = jax.lax.bitcast_convert_type((vbits << 16) | vbits,
                                        jnp.int32).reshape(_EBLK, 128)
    x1 = _spmm(emb.astype(jnp.bfloat16), row2, col2, val2)
    x2 = _spmm(x1, row2, col2, val2)
    hyper = _avg3(emb, x1, x2)
    b32 = base.astype(jnp.int32)
    hidx = b32[:, :_K].reshape(_TBLK, 128)
    ridx = b32[:, _K:2 * _K].reshape(_TBLK, 128)
    tidx = b32[:, 2 * _K:].reshape(_TBLK, 128)
    scores = _dec(hyper, emb, hidx, ridx, tidx)
    loss = _loss(scores.reshape(_B, _K), ground_truth.astype(jnp.float32), emb)
    return loss[0, 0]


# speedup vs baseline: 9.1604x; 1.0862x over previous
import jax
import jax.numpy as jnp
from jax import lax
from jax.experimental import pallas as pl
from jax.experimental.pallas import tpu as pltpu
from jax.experimental.pallas import tpu_sc as plsc

_N = 50000
_EMB = 64
_B = 4096
_K = 32
_NNZ = 800000
_NC = 2
_NS = 16
_RPS = _N // _NC
_EBLK = _NNZ // 128
_QK = (_EBLK + _NS - 1) // _NS
_QK2 = (_QK + 1) // 2
_ZCH = 200
_NCH = _RPS // _ZCH
_NCHK = (_NCH + _NS - 1) // _NS
_TBLK = (_B * _K) // 128

_mesh = plsc.VectorSubcoreMesh(core_axis_name="c", subcore_axis_name="s")
_sc_params = pltpu.CompilerParams(use_tc_tiling_on_sc=False,
                                  needs_layout_passes=False)


def _spmm_body(x_hbm, row_hbm, col_hbm, val_hbm, out_hbm,
               acc, erow, ecol, eval_, lrow, grows,
               msem0, msem1, gsem0, gsem1, ssem0, ssem1, zsem):
    c = lax.axis_index("c")
    s = lax.axis_index("s")
    lo = c * _RPS
    msem = (msem0, msem1)
    gsem = (gsem0, gsem1)
    ssem = (ssem0, ssem1)
    z32 = jnp.zeros((32,), jnp.bfloat16)

    def _zb(r, cy):
        for cc in range(_EMB // 32):
            grows[r, pl.ds(cc * 32, 32)] = z32
        return cy
    lax.fori_loop(0, _ZCH, _zb, 0)

    def _zc(k, cy):
        j = s + _NS * k

        @pl.when(j < _NCH)
        def _():
            pltpu.async_copy(grows.at[pl.ds(0, _ZCH)],
                             acc.at[pl.ds(j * _ZCH, _ZCH)], zsem)
        return cy
    lax.fori_loop(0, _NCHK, _zc, 0)

    def _zw(k, cy):
        j = s + _NS * k

        @pl.when(j < _NCH)
        def _():
            pltpu.make_async_copy(grows.at[pl.ds(0, _ZCH)],
                                  acc.at[pl.ds(j * _ZCH, _ZCH)], zsem).wait()
        return cy
    lax.fori_loop(0, _NCHK, _zw, 0)
    plsc.subcore_barrier()

    def _q(k):
        return s + _NS * k

    def _start_meta(k, b):
        @pl.when(_q(k) < _EBLK)
        def _():
            q = _q(k)
            pltpu.async_copy(row_hbm.at[pl.ds(q, 1)],
                             erow.at[pl.ds(b, 1)], msem[b])
            pltpu.async_copy(col_hbm.at[pl.ds(q, 1)],
                             ecol.at[pl.ds(b, 1)], msem[b])
            pltpu.async_copy(val_hbm.at[pl.ds(q, 1)],
                             eval_.at[pl.ds(b, 1)], msem[b])

    def _wait_meta(k, b):
        @pl.when(_q(k) < _EBLK)
        def _():
            pltpu.make_async_copy(row_hbm.at[pl.ds(0, 1)],
                                  erow.at[pl.ds(b, 1)], msem[b]).wait()
            pltpu.make_async_copy(col_hbm.at[pl.ds(0, 1)],
                                  ecol.at[pl.ds(b, 1)], msem[b]).wait()
            pltpu.make_async_copy(val_hbm.at[pl.ds(0, 1)],
                                  eval_.at[pl.ds(b, 1)], msem[b]).wait()

    def _start_gather(k, b):
        @pl.when(_q(k) < _EBLK)
        def _():
            pltpu.async_copy(x_hbm.at[ecol.at[b]],
                             grows.at[pl.ds(b * 128, 128)], gsem[b])

    def _wait_gather(k, b):
        @pl.when(_q(k) < _EBLK)
        def _():
            pltpu.make_async_copy(
                x_hbm.at[pl.ds(0, 128)],
                grows.at[pl.ds(b * 128, 128)], gsem[b]).wait()

    def _compute(k, b):
        @pl.when(_q(k) < _EBLK)
        def _():
            def _msk(g, cy2):
                off = pl.multiple_of(g * 16, 16)
                rv = erow[b, pl.ds(off, 16)]
                vv = eval_[b, pl.ds(off, 16)]
                inr = (rv >= lo) & (rv < lo + _RPS)
                lrow[b, pl.ds(off, 16)] = jnp.clip(rv - lo, 0, _RPS - 1)
                eval_[b, pl.ds(off, 16)] = jnp.where(inr, vv, 0)
                return cy2
            lax.fori_loop(0, 8, _msk, 0)

            def _edge(g, cy2):
                off = pl.multiple_of(g * 16, 16)
                vv = eval_[b, pl.ds(off, 16)]
                for j in range(16):
                    vmv = plsc.bitcast(jnp.full((16,), vv[j], jnp.int32),
                                       jnp.bfloat16)
                    r = b * 128 + off + j
                    for cc in range(_EMB // 32):
                        sl = pl.ds(cc * 32, 32)
                        grows[r, sl] = grows[r, sl] * vmv
                return cy2
            lax.fori_loop(0, 8, _edge, 0)

    def _start_scatter(k, b):
        @pl.when(_q(k) < _EBLK)
        def _():
            pltpu.async_copy(grows.at[pl.ds(b * 128, 128)],
                             acc.at[lrow.at[b]], ssem[b], add=True)

    def _wait_scatter(k, b):
        @pl.when(jnp.logical_and(k >= 0, _q(k) < _EBLK))
        def _():
            pltpu.make_async_copy(
                x_hbm.at[pl.ds(0, 128)],
                grows.at[pl.ds(b * 128, 128)], ssem[b]).wait()

    _start_meta(0, 0)
    _start_meta(1, 1)
    _wait_meta(0, 0)
    _start_gather(0, 0)

    def _body(t, cy):
        for b in range(2):
            k = 2 * t + b
            nb = 1 - b
            _wait_gather(k, b)
            _wait_meta(k + 1, nb)
            _wait_scatter(k - 1, nb)
            _start_gather(k + 1, nb)
            _compute(k, b)
            _start_meta(k + 2, b)
            _start_scatter(k, b)
        return cy
    lax.fori_loop(0, _QK2, _body, 0)
    _wait_scatter(2 * _QK2 - 1, 1)
    plsc.subcore_barrier()

    def _wo(k, cy):
        j = s + _NS * k

        @pl.when(j < _NCH)
        def _():
            pltpu.async_copy(acc.at[pl.ds(j * _ZCH, _ZCH)],
                             out_hbm.at[pl.ds(lo + j * _ZCH, _ZCH)], zsem)
        return cy
    lax.fori_loop(0, _NCHK, _wo, 0)

    def _wow(k, cy):
        j = s + _NS * k

        @pl.when(j < _NCH)
        def _():
            pltpu.make_async_copy(acc.at[pl.ds(j * _ZCH, _ZCH)],
                                  out_hbm.at[pl.ds(lo + j * _ZCH, _ZCH)],
                                  zsem).wait()
        return cy
    lax.fori_loop(0, _NCHK, _wow, 0)


def _spmm(x, row2, col2, val2):
    f = pl.kernel(
        _spmm_body,
        out_type=jax.ShapeDtypeStruct((_N, _EMB), jnp.bfloat16),
        mesh=_mesh,
        scratch_types=[
            pltpu.VMEM_SHARED((_RPS, _EMB), jnp.bfloat16),
            pltpu.VMEM((2, 128), jnp.int32),
            pltpu.VMEM((2, 128), jnp.int32),
            pltpu.VMEM((2, 128), jnp.int32),
            pltpu.VMEM((2, 128), jnp.int32),
            pltpu.VMEM((256, _EMB), jnp.bfloat16),
            pltpu.SemaphoreType.DMA,
            pltpu.SemaphoreType.DMA,
            pltpu.SemaphoreType.DMA,
            pltpu.SemaphoreType.DMA,
            pltpu.SemaphoreType.DMA,
            pltpu.SemaphoreType.DMA,
            pltpu.SemaphoreType.DMA,
        ],
        compiler_params=_sc_params,
    )
    return f(x, row2, col2, val2)


_DBT = _TBLK // (_NC * _NS)


def _dec_body(hyper, emb, hidx, ridx, tidx, out_hbm,
              ihb, irb, itb, gh, gr, gt, sbuf,
              isem0, isem1, gsem0, gsem1, osem0, osem1):
    c = lax.axis_index("c")
    s = lax.axis_index("s")
    wid = s * _NC + c
    isem = (isem0, isem1)
    gsem = (gsem0, gsem1)
    osem = (osem0, osem1)
    lanes = lax.iota(jnp.int32, 16)

    def _start_idx(k, b):
        @pl.when(jnp.logical_and(k >= 0, k < _DBT))
        def _():
            blk = wid * _DBT + k
            pltpu.async_copy(hidx.at[pl.ds(blk, 1)],
                             ihb.at[pl.ds(b, 1)], isem[b])
            pltpu.async_copy(ridx.at[pl.ds(blk, 1)],
                             irb.at[pl.ds(b, 1)], isem[b])
            pltpu.async_copy(tidx.at[pl.ds(blk, 1)],
                             itb.at[pl.ds(b, 1)], isem[b])

    def _wait_idx(k, b):
        @pl.when(jnp.logical_and(k >= 0, k < _DBT))
        def _():
            for ref in (ihb, irb, itb):
                pltpu.make_async_copy(hidx.at[pl.ds(0, 1)],
                                      ref.at[pl.ds(b, 1)], isem[b]).wait()

    def _start_gather(k, b):
        @pl.when(jnp.logical_and(k >= 0, k < _DBT))
        def _():
            pltpu.async_copy(hyper.at[ihb.at[b]],
                             gh.at[pl.ds(b * 128, 128)], gsem[b])
            pltpu.async_copy(emb.at[irb.at[b]],
                             gr.at[pl.ds(b * 128, 128)], gsem[b])
            pltpu.async_copy(hyper.at[itb.at[b]],
                             gt.at[pl.ds(b * 128, 128)], gsem[b])

    def _wait_gather(k, b):
        @pl.when(jnp.logical_and(k >= 0, k < _DBT))
        def _():
            for ref in (gh, gr, gt):
                pltpu.make_async_copy(hyper.at[pl.ds(0, 128)],
                                      ref.at[pl.ds(b * 128, 128)],
                                      gsem[b]).wait()

    def _compute(k, b):
        @pl.when(k < _DBT)
        def _():
            def _grp(g, cy2):
                off = pl.multiple_of(g * 16, 16)
                svec = jnp.zeros((16,), jnp.float32)
                for j in range(16):
                    e = b * 128 + off + j
                    acc = jnp.zeros((16,), jnp.float32)
                    for cc in range(_EMB // 32):
                        slre = pl.ds(cc * 16, 16)
                        slim = pl.ds(_EMB // 2 + cc * 16, 16)
                        hr = gh[e, slre]
                        hi = gh[e, slim]
                        rr = gr[e, slre]
                        ri = gr[e, slim]
                        tr = gt[e, slre]
                        ti = gt[e, slim]
                        acc = (acc + (hr * rr - hi * ri) * tr
                               + (hr * ri + hi * rr) * ti)
                    tot = jnp.sum(acc)
                    svec = jnp.where(lanes == j, tot, svec)
                sbuf[b, pl.ds(off, 16)] = svec
                return cy2
            lax.fori_loop(0, 8, _grp, 0)

    def _start_out(k, b):
        @pl.when(k < _DBT)
        def _():
            blk = wid * _DBT + k
            pltpu.async_copy(sbuf.at[b], out_hbm.at[pl.ds(blk * 128, 128)],
                             osem[b])

    def _wait_out(k, b):
        @pl.when(jnp.logical_and(k >= 0, k < _DBT))
        def _():
            pltpu.make_async_copy(sbuf.at[b],
                                  out_hbm.at[pl.ds(0, 128)], osem[b]).wait()

    _start_idx(0, 0)
    _start_idx(1, 1)
    _wait_idx(0, 0)
    _start_gather(0, 0)

    def _body(t, cy):
        for b in range(2):
            k = 2 * t + b
            nb = 1 - b
            _wait_gather(k, b)
            _wait_idx(k + 1, nb)
            _start_gather(k + 1, nb)
            _wait_out(k - 2, b)
            _compute(k, b)
            _start_idx(k + 2, b)
            _start_out(k, b)
        return cy
    lax.fori_loop(0, _DBT // 2, _body, 0)
    _wait_out(_DBT - 2, 0)
    _wait_out(_DBT - 1, 1)


def _dec(hyper, emb, hidx, ridx, tidx):
    f = pl.kernel(
        _dec_body,
        out_type=jax.ShapeDtypeStruct((_B * _K,), jnp.float32),
        mesh=_mesh,
        scratch_types=[
            pltpu.VMEM((2, 128), jnp.int32),
            pltpu.VMEM((2, 128), jnp.int32),
            pltpu.VMEM((2, 128), jnp.int32),
            pltpu.VMEM((256, _EMB), jnp.float32),
            pltpu.VMEM((256, _EMB), jnp.float32),
            pltpu.VMEM((256, _EMB), jnp.float32),
            pltpu.VMEM((2, 128), jnp.float32),
            pltpu.SemaphoreType.DMA,
            pltpu.SemaphoreType.DMA,
            pltpu.SemaphoreType.DMA,
            pltpu.SemaphoreType.DMA,
            pltpu.SemaphoreType.DMA,
            pltpu.SemaphoreType.DMA,
        ],
        compiler_params=_sc_params,
    )
    return f(hyper, emb, hidx, ridx, tidx)


def _avg_body(a_ref, b_ref, c_ref, o_ref):
    o_ref[...] = ((a_ref[...].astype(jnp.float32)
                   + b_ref[...].astype(jnp.float32)
                   + c_ref[...].astype(jnp.float32))
                  * jnp.float32(1.0 / 3.0))


def _avg3(a, b, c):
    g = 25
    blk = _N // g
    return pl.pallas_call(
        _avg_body,
        out_shape=jax.ShapeDtypeStruct((_N, _EMB), jnp.float32),
        grid=(g,),
        in_specs=[pl.BlockSpec((blk, _EMB), lambda i: (i, 0))] * 3,
        out_specs=pl.BlockSpec((blk, _EMB), lambda i: (i, 0)),
    )(a, b, c)


def _loss_body(sc_ref, gt_ref, emb_ref, o_ref):
    sv = jax.nn.sigmoid(sc_ref[...])
    gv = gt_ref[...]
    eps = jnp.float32(1e-7)
    bce = -jnp.mean(gv * jnp.log(sv + eps)
                    + (1.0 - gv) * jnp.log(1.0 - sv + eps))
    ev = emb_ref[...]
    ae = jnp.abs(ev)
    regu = jnp.float32(0.01) * jnp.sum(ae * ae * ae)
    o_ref[0, 0] = bce + regu


def _loss(scores, gt, emb):
    return pl.pallas_call(
        _loss_body,
        out_shape=jax.ShapeDtypeStruct((1, 1), jnp.float32),
        out_specs=pl.BlockSpec(memory_space=pltpu.SMEM),
    )(scores, gt, emb)


def kernel(embedding, adj_row, adj_col, adj_val, base, ground_truth):
    emb = embedding.astype(jnp.float32)
    row2 = adj_row.astype(jnp.int32).reshape(_EBLK, 128)
    col2 = adj_col.astype(jnp.int32).reshape(_EBLK, 128)
    vbits = jax.lax.bitcast_convert_type(adj_val.astype(jnp.bfloat16),
                                         jnp.uint16).astype(jnp.uint32)
    val2 = jax.lax.bitcast_convert_type((vbits << 16) | vbits,
                                        jnp.int32).reshape(_EBLK, 128)
    x1 = _spmm(emb.astype(jnp.bfloat16), row2, col2, val2)
    x2 = _spmm(x1, row2, col2, val2)
    hyper = _avg3(emb, x1, x2)
    b32 = base.astype(jnp.int32)
    hidx = b32[:, :_K].reshape(_TBLK, 128)
    ridx = b32[:, _K:2 * _K].reshape(_TBLK, 128)
    tidx = b32[:, 2 * _K:].reshape(_TBLK, 128)
    scores = _dec(hyper, emb, hidx, ridx, tidx)
    loss = _loss(scores.reshape(_B, _K), ground_truth.astype(jnp.float32), emb)
    return loss[0, 0]


# speedup vs baseline: 9.1644x; 1.0004x over previous
import jax
import jax.numpy as jnp
from jax import lax
from jax.experimental import pallas as pl
from jax.experimental.pallas import tpu as pltpu
from jax.experimental.pallas import tpu_sc as plsc

_N = 50000
_EMB = 64
_B = 4096
_K = 32
_NNZ = 800000
_NC = 2
_NS = 16
_RPS = _N // _NC
_EBLK = _NNZ // 128
_QK = (_EBLK + _NS - 1) // _NS
_QK2 = (_QK + 1) // 2
_ZCH = 200
_NCH = _RPS // _ZCH
_NCHK = (_NCH + _NS - 1) // _NS
_TBLK = (_B * _K) // 128

_mesh = plsc.VectorSubcoreMesh(core_axis_name="c", subcore_axis_name="s")
_sc_params = pltpu.CompilerParams(use_tc_tiling_on_sc=False,
                                  needs_layout_passes=False)


def _spmm_body(x_hbm, row_hbm, col_hbm, val_hbm, out_hbm,
               acc, erow, ecol, eval_, lrow, grows,
               msem0, msem1, gsem0, gsem1, ssem0, ssem1, zsem):
    c = lax.axis_index("c")
    s = lax.axis_index("s")
    lo = c * _RPS
    msem = (msem0, msem1)
    gsem = (gsem0, gsem1)
    ssem = (ssem0, ssem1)
    z32 = jnp.zeros((32,), jnp.bfloat16)

    def _zb(r, cy):
        for cc in range(_EMB // 32):
            grows[r, pl.ds(cc * 32, 32)] = z32
        return cy
    lax.fori_loop(0, _ZCH, _zb, 0)

    def _zc(k, cy):
        j = s + _NS * k

        @pl.when(j < _NCH)
        def _():
            pltpu.async_copy(grows.at[pl.ds(0, _ZCH)],
                             acc.at[pl.ds(j * _ZCH, _ZCH)], zsem)
        return cy
    lax.fori_loop(0, _NCHK, _zc, 0)

    def _zw(k, cy):
        j = s + _NS * k

        @pl.when(j < _NCH)
        def _():
            pltpu.make_async_copy(grows.at[pl.ds(0, _ZCH)],
                                  acc.at[pl.ds(j * _ZCH, _ZCH)], zsem).wait()
        return cy
    lax.fori_loop(0, _NCHK, _zw, 0)
    plsc.subcore_barrier()

    def _q(k):
        return s + _NS * k

    def _start_meta(k, b):
        @pl.when(_q(k) < _EBLK)
        def _():
            q = _q(k)
            pltpu.async_copy(row_hbm.at[pl.ds(q, 1)],
                             erow.at[pl.ds(b, 1)], msem[b])
            pltpu.async_copy(col_hbm.at[pl.ds(q, 1)],
                             ecol.at[pl.ds(b, 1)], msem[b])
            pltpu.async_copy(val_hbm.at[pl.ds(q, 1)],
                             eval_.at[pl.ds(b, 1)], msem[b])

    def _wait_meta(k, b):
        @pl.when(_q(k) < _EBLK)
        def _():
            pltpu.make_async_copy(row_hbm.at[pl.ds(0, 1)],
                                  erow.at[pl.ds(b, 1)], msem[b]).wait()
            pltpu.make_async_copy(col_hbm.at[pl.ds(0, 1)],
                                  ecol.at[pl.ds(b, 1)], msem[b]).wait()
            pltpu.make_async_copy(val_hbm.at[pl.ds(0, 1)],
                                  eval_.at[pl.ds(b, 1)], msem[b]).wait()

    def _start_gather(k, b):
        @pl.when(_q(k) < _EBLK)
        def _():
            pltpu.async_copy(x_hbm.at[ecol.at[b]],
                             grows.at[pl.ds(b * 128, 128)], gsem[b])

    def _wait_gather(k, b):
        @pl.when(_q(k) < _EBLK)
        def _():
            pltpu.make_async_copy(
                x_hbm.at[pl.ds(0, 128)],
                grows.at[pl.ds(b * 128, 128)], gsem[b]).wait()

    def _compute(k, b):
        @pl.when(_q(k) < _EBLK)
        def _():
            def _msk(g, cy2):
                off = pl.multiple_of(g * 16, 16)
                rv = erow[b, pl.ds(off, 16)]
                vv = eval_[b, pl.ds(off, 16)]
                inr = (rv >= lo) & (rv < lo + _RPS)
                lrow[b, pl.ds(off, 16)] = jnp.clip(rv - lo, 0, _RPS - 1)
                eval_[b, pl.ds(off, 16)] = jnp.where(inr, vv, 0)
                return cy2
            lax.fori_loop(0, 8, _msk, 0)

            def _edge(g, cy2):
                off = pl.multiple_of(g * 16, 16)
                vv = eval_[b, pl.ds(off, 16)]
                for j in range(16):
                    splat = lax.gather(
                        vv, jnp.full((16, 1), j, jnp.int32),
                        lax.GatherDimensionNumbers(
                            offset_dims=(), collapsed_slice_dims=(0,),
                            start_index_map=(0,)),
                        (1,), mode=lax.GatherScatterMode.PROMISE_IN_BOUNDS)
                    vmv = plsc.bitcast(splat, jnp.bfloat16)
                    r = b * 128 + off + j
                    for cc in range(_EMB // 32):
                        sl = pl.ds(cc * 32, 32)
                        grows[r, sl] = grows[r, sl] * vmv
                return cy2
            lax.fori_loop(0, 8, _edge, 0)

    def _start_scatter(k, b):
        @pl.when(_q(k) < _EBLK)
        def _():
            pltpu.async_copy(grows.at[pl.ds(b * 128, 128)],
                             acc.at[lrow.at[b]], ssem[b], add=True)

    def _wait_scatter(k, b):
        @pl.when(jnp.logical_and(k >= 0, _q(k) < _EBLK))
        def _():
            pltpu.make_async_copy(
                x_hbm.at[pl.ds(0, 128)],
                grows.at[pl.ds(b * 128, 128)], ssem[b]).wait()

    _start_meta(0, 0)
    _start_meta(1, 1)
    _wait_meta(0, 0)
    _start_gather(0, 0)

    def _body(t, cy):
        for b in range(2):
            k = 2 * t + b
            nb = 1 - b
            _wait_gather(k, b)
            _wait_meta(k + 1, nb)
            _wait_scatter(k - 1, nb)
            _start_gather(k + 1, nb)
            _compute(k, b)
            _start_meta(k + 2, b)
            _start_scatter(k, b)
        return cy
    lax.fori_loop(0, _QK2, _body, 0)
    _wait_scatter(2 * _QK2 - 1, 1)
    plsc.subcore_barrier()

    def _wo(k, cy):
        j = s + _NS * k

        @pl.when(j < _NCH)
        def _():
            pltpu.async_copy(acc.at[pl.ds(j * _ZCH, _ZCH)],
                             out_hbm.at[pl.ds(lo + j * _ZCH, _ZCH)], zsem)
        return cy
    lax.fori_loop(0, _NCHK, _wo, 0)

    def _wow(k, cy):
        j = s + _NS * k

        @pl.when(j < _NCH)
        def _():
            pltpu.make_async_copy(acc.at[pl.ds(j * _ZCH, _ZCH)],
                                  out_hbm.at[pl.ds(lo + j * _ZCH, _ZCH)],
                                  zsem).wait()
        return cy
    lax.fori_loop(0, _NCHK, _wow, 0)


def _spmm(x, row2, col2, val2):
    f = pl.kernel(
        _spmm_body,
        out_type=jax.ShapeDtypeStruct((_N, _EMB), jnp.bfloat16),
        mesh=_mesh,
        scratch_types=[
            pltpu.VMEM_SHARED((_RPS, _EMB), jnp.bfloat16),
            pltpu.VMEM((2, 128), jnp.int32),
            pltpu.VMEM((2, 128), jnp.int32),
            pltpu.VMEM((2, 128), jnp.int32),
            pltpu.VMEM((2, 128), jnp.int32),
            pltpu.VMEM((256, _EMB), jnp.bfloat16),
            pltpu.SemaphoreType.DMA,
            pltpu.SemaphoreType.DMA,
            pltpu.SemaphoreType.DMA,
            pltpu.SemaphoreType.DMA,
            pltpu.SemaphoreType.DMA,
            pltpu.SemaphoreType.DMA,
            pltpu.SemaphoreType.DMA,
        ],
        compiler_params=_sc_params,
    )
    return f(x, row2, col2, val2)


_DBT = _TBLK // (_NC * _NS)


def _dec_body(hyper, emb, hidx, ridx, tidx, out_hbm,
              ihb, irb, itb, gh, gr, gt, sbuf,
              isem0, isem1, gsem0, gsem1, osem0, osem1):
    c = lax.axis_index("c")
    s = lax.axis_index("s")
    wid = s * _NC + c
    isem = (isem0, isem1)
    gsem = (gsem0, gsem1)
    osem = (osem0, osem1)
    lanes = lax.iota(jnp.int32, 16)

    def _start_idx(k, b):
        @pl.when(jnp.logical_and(k >= 0, k < _DBT))
        def _():
            blk = wid * _DBT + k
            pltpu.async_copy(hidx.at[pl.ds(blk, 1)],
                             ihb.at[pl.ds(b, 1)], isem[b])
            pltpu.async_copy(ridx.at[pl.ds(blk, 1)],
                             irb.at[pl.ds(b, 1)], isem[b])
            pltpu.async_copy(tidx.at[pl.ds(blk, 1)],
                             itb.at[pl.ds(b, 1)], isem[b])

    def _wait_idx(k, b):
        @pl.when(jnp.logical_and(k >= 0, k < _DBT))
        def _():
            for ref in (ihb, irb, itb):
                pltpu.make_async_copy(hidx.at[pl.ds(0, 1)],
                                      ref.at[pl.ds(b, 1)], isem[b]).wait()

    def _start_gather(k, b):
        @pl.when(jnp.logical_and(k >= 0, k < _DBT))
        def _():
            pltpu.async_copy(hyper.at[ihb.at[b]],
                             gh.at[pl.ds(b * 128, 128)], gsem[b])
            pltpu.async_copy(emb.at[irb.at[b]],
                             gr.at[pl.ds(b * 128, 128)], gsem[b])
            pltpu.async_copy(hyper.at[itb.at[b]],
                             gt.at[pl.ds(b * 128, 128)], gsem[b])

    def _wait_gather(k, b):
        @pl.when(jnp.logical_and(k >= 0, k < _DBT))
        def _():
            for ref in (gh, gr, gt):
                pltpu.make_async_copy(hyper.at[pl.ds(0, 128)],
                                      ref.at[pl.ds(b * 128, 128)],
                                      gsem[b]).wait()

    def _compute(k, b):
        @pl.when(k < _DBT)
        def _():
            def _grp(g, cy2):
                off = pl.multiple_of(g * 16, 16)
                svec = jnp.zeros((16,), jnp.float32)
                for j in range(16):
                    e = b * 128 + off + j
                    acc = jnp.zeros((16,), jnp.float32)
                    for cc in range(_EMB // 32):
                        slre = pl.ds(cc * 16, 16)
                        slim = pl.ds(_EMB // 2 + cc * 16, 16)
                        hr = gh[e, slre]
                        hi = gh[e, slim]
                        rr = gr[e, slre]
                        ri = gr[e, slim]
                        tr = gt[e, slre]
                        ti = gt[e, slim]
                        acc = (acc + (hr * rr - hi * ri) * tr
                               + (hr * ri + hi * rr) * ti)
                    tot = jnp.sum(acc)
                    svec = jnp.where(lanes == j, tot, svec)
                sbuf[b, pl.ds(off, 16)] = svec
                return cy2
            lax.fori_loop(0, 8, _grp, 0)

    def _start_out(k, b):
        @pl.when(k < _DBT)
        def _():
            blk = wid * _DBT + k
            pltpu.async_copy(sbuf.at[b], out_hbm.at[pl.ds(blk * 128, 128)],
                             osem[b])

    def _wait_out(k, b):
        @pl.when(jnp.logical_and(k >= 0, k < _DBT))
        def _():
            pltpu.make_async_copy(sbuf.at[b],
                                  out_hbm.at[pl.ds(0, 128)], osem[b]).wait()

    _start_idx(0, 0)
    _start_idx(1, 1)
    _wait_idx(0, 0)
    _start_gather(0, 0)

    def _body(t, cy):
        for b in range(2):
            k = 2 * t + b
            nb = 1 - b
            _wait_gather(k, b)
            _wait_idx(k + 1, nb)
            _start_gather(k + 1, nb)
            _wait_out(k - 2, b)
            _compute(k, b)
            _start_idx(k + 2, b)
            _start_out(k, b)
        return cy
    lax.fori_loop(0, _DBT // 2, _body, 0)
    _wait_out(_DBT - 2, 0)
    _wait_out(_DBT - 1, 1)


def _dec(hyper, emb, hidx, ridx, tidx):
    f = pl.kernel(
        _dec_body,
        out_type=jax.ShapeDtypeStruct((_B * _K,), jnp.float32),
        mesh=_mesh,
        scratch_types=[
            pltpu.VMEM((2, 128), jnp.int32),
            pltpu.VMEM((2, 128), jnp.int32),
            pltpu.VMEM((2, 128), jnp.int32),
            pltpu.VMEM((256, _EMB), jnp.float32),
            pltpu.VMEM((256, _EMB), jnp.float32),
            pltpu.VMEM((256, _EMB), jnp.float32),
            pltpu.VMEM((2, 128), jnp.float32),
            pltpu.SemaphoreType.DMA,
            pltpu.SemaphoreType.DMA,
            pltpu.SemaphoreType.DMA,
            pltpu.SemaphoreType.DMA,
            pltpu.SemaphoreType.DMA,
            pltpu.SemaphoreType.DMA,
        ],
        compiler_params=_sc_params,
    )
    return f(hyper, emb, hidx, ridx, tidx)


def _avg_body(a_ref, b_ref, c_ref, o_ref):
    o_ref[...] = ((a_ref[...].astype(jnp.float32)
                   + b_ref[...].astype(jnp.float32)
                   + c_ref[...].astype(jnp.float32))
                  * jnp.float32(1.0 / 3.0))


def _avg3(a, b, c):
    g = 25
    blk = _N // g
    return pl.pallas_call(
        _avg_body,
        out_shape=jax.ShapeDtypeStruct((_N, _EMB), jnp.float32),
        grid=(g,),
        in_specs=[pl.BlockSpec((blk, _EMB), lambda i: (i, 0))] * 3,
        out_specs=pl.BlockSpec((blk, _EMB), lambda i: (i, 0)),
    )(a, b, c)


def _loss_body(sc_ref, gt_ref, emb_ref, o_ref):
    sv = jax.nn.sigmoid(sc_ref[...])
    gv = gt_ref[...]
    eps = jnp.float32(1e-7)
    bce = -jnp.mean(gv * jnp.log(sv + eps)
                    + (1.0 - gv) * jnp.log(1.0 - sv + eps))
    ev = emb_ref[...]
    ae = jnp.abs(ev)
    regu = jnp.float32(0.01) * jnp.sum(ae * ae * ae)
    o_ref[0, 0] = bce + regu


def _loss(scores, gt, emb):
    return pl.pallas_call(
        _loss_body,
        out_shape=jax.ShapeDtypeStruct((1, 1), jnp.float32),
        out_specs=pl.BlockSpec(memory_space=pltpu.SMEM),
    )(scores, gt, emb)


def kernel(embedding, adj_row, adj_col, adj_val, base, ground_truth):
    emb = embedding.astype(jnp.float32)
    row2 = adj_row.astype(jnp.int32).reshape(_EBLK, 128)
    col2 = adj_col.astype(jnp.int32).reshape(_EBLK, 128)
    vbits = jax.lax.bitcast_convert_type(adj_val.astype(jnp.bfloat16),
                                         jnp.uint16).astype(jnp.uint32)
    val2 = jax.lax.bitcast_convert_type((vbits << 16) | vbits,
                                        jnp.int32).reshape(_EBLK, 128)
    x1 = _spmm(emb.astype(jnp.bfloat16), row2, col2, val2)
    x2 = _spmm(x1, row2, col2, val2)
    hyper = _avg3(emb, x1, x2)
    b32 = base.astype(jnp.int32)
    hidx = b32[:, :_K].reshape(_TBLK, 128)
    ridx = b32[:, _K:2 * _K].reshape(_TBLK, 128)
    tidx = b32[:, 2 * _K:].reshape(_TBLK, 128)
    scores = _dec(hyper, emb, hidx, ridx, tidx)
    loss = _loss(scores.reshape(_B, _K), ground_truth.astype(jnp.float32), emb)
    return loss[0, 0]
